# baseline (speedup 1.0000x reference)
import functools

import jax
import jax.numpy as jnp
from jax import lax
from jax.experimental import pallas as pl
from jax.experimental.pallas import tpu as pltpu
from jax.experimental.pallas import tpu_sc as plsc

_CHUNK = 512


@functools.lru_cache(maxsize=None)
def _make_gather(n_flat: int, vocab: int, d: int):
    info = plsc.get_sparse_core_info()
    nw = info.num_cores * info.num_subcores
    assert n_flat % (nw * _CHUNK) == 0
    b_per_w = n_flat // nw
    n_chunks = b_per_w // _CHUNK
    mesh = plsc.VectorSubcoreMesh(core_axis_name="c", subcore_axis_name="s")

    @functools.partial(
        pl.kernel,
        mesh=mesh,
        out_type=jax.ShapeDtypeStruct((n_flat, d), jnp.float32),
        scratch_types=[
            pltpu.VMEM((_CHUNK,), jnp.int32),
            pltpu.VMEM((_CHUNK, d), jnp.float32),
            pltpu.SemaphoreType.DMA,
        ],
        compiler_params=pltpu.CompilerParams(use_tc_tiling_on_sc=False),
    )
    def gather_kernel(idx_hbm, table_hbm, out_hbm, idx_v, rows_v, sem):
        wid = lax.axis_index("s") * info.num_cores + lax.axis_index("c")
        base = wid * b_per_w

        def body(ci, carry):
            row0 = base + ci * _CHUNK
            pltpu.sync_copy(idx_hbm.at[pl.ds(row0, _CHUNK)], idx_v)
            pltpu.async_copy(table_hbm.at[idx_v], rows_v, sem).wait()
            pltpu.sync_copy(rows_v, out_hbm.at[pl.ds(row0, _CHUNK)])
            return carry

        lax.fori_loop(0, n_chunks, body, 0)

    return gather_kernel


def kernel(input, table):
    b, l = input.shape
    vocab, d = table.shape
    flat_idx = input.reshape(b * l)
    out = _make_gather(b * l, vocab, d)(flat_idx, table)
    return out.reshape(b, l, d)


# speedup vs baseline: 1.0385x; 1.0385x over previous
import functools

import jax
import jax.numpy as jnp
from jax import lax
from jax.experimental import pallas as pl
from jax.experimental.pallas import tpu as pltpu
from jax.experimental.pallas import tpu_sc as plsc

_CHUNK = 400
_NBUF = 4


@functools.lru_cache(maxsize=None)
def _make_gather(n_flat: int, vocab: int, d: int):
    info = plsc.get_sparse_core_info()
    nw = info.num_cores * info.num_subcores
    assert n_flat % (nw * _CHUNK * _NBUF) == 0
    b_per_w = n_flat // nw
    n_chunks = b_per_w // _CHUNK
    n_steps = n_chunks // _NBUF
    mesh = plsc.VectorSubcoreMesh(core_axis_name="c", subcore_axis_name="s")

    @functools.partial(
        pl.kernel,
        mesh=mesh,
        out_type=jax.ShapeDtypeStruct((n_flat, d), jnp.float32),
        scratch_types=[
            pltpu.VMEM((_NBUF, _CHUNK), jnp.int32),
            pltpu.VMEM((_NBUF, _CHUNK, d), jnp.float32),
            pltpu.SemaphoreType.DMA((_NBUF,)),
            pltpu.SemaphoreType.DMA((_NBUF,)),
        ],
        compiler_params=pltpu.CompilerParams(use_tc_tiling_on_sc=False),
    )
    def gather_kernel(idx_hbm, table_hbm, out_hbm, idx_v, rows_v, gsem, ssem):
        wid = lax.axis_index("s") * info.num_cores + lax.axis_index("c")
        base = wid * b_per_w

        def start_gather(ci, b):
            row0 = base + ci * _CHUNK
            pltpu.sync_copy(idx_hbm.at[pl.ds(row0, _CHUNK)], idx_v.at[b])
            pltpu.async_copy(table_hbm.at[idx_v.at[b]], rows_v.at[b],
                             gsem.at[b])

        def start_scatter(ci, b):
            row0 = base + ci * _CHUNK
            pltpu.async_copy(rows_v.at[b], out_hbm.at[pl.ds(row0, _CHUNK)],
                             ssem.at[b])

        for b in range(_NBUF):
            start_gather(b, b)

        def body(g, carry):
            for b in range(_NBUF):
                ci = g * _NBUF + b
                pltpu.make_async_copy(
                    table_hbm.at[idx_v.at[b]], rows_v.at[b], gsem.at[b]
                ).wait()
                start_scatter(ci, b)
                pltpu.make_async_copy(
                    rows_v.at[b], out_hbm.at[pl.ds(base + ci * _CHUNK, _CHUNK)],
                    ssem.at[b],
                ).wait()
                start_gather(ci + _NBUF, b)
            return carry

        lax.fori_loop(0, n_steps - 1, body, 0)

        for b in range(_NBUF):
            ci = (n_steps - 1) * _NBUF + b
            pltpu.make_async_copy(
                table_hbm.at[idx_v.at[b]], rows_v.at[b], gsem.at[b]
            ).wait()
            start_scatter(ci, b)
        for b in range(_NBUF):
            ci = (n_steps - 1) * _NBUF + b
            pltpu.make_async_copy(
                rows_v.at[b], out_hbm.at[pl.ds(base + ci * _CHUNK, _CHUNK)],
                ssem.at[b],
            ).wait()

    return gather_kernel


def kernel(input, table):
    b, l = input.shape
    vocab, d = table.shape
    flat_idx = input.reshape(b * l)
    out = _make_gather(b * l, vocab, d)(flat_idx, table)
    return out.reshape(b, l, d)


# speedup vs baseline: 1.4218x; 1.3691x over previous
import functools

import jax
import jax.numpy as jnp
from jax import lax
from jax.experimental import pallas as pl
from jax.experimental.pallas import tpu as pltpu
from jax.experimental.pallas import tpu_sc as plsc

_CHUNK = 512
_NBUF = 2


@functools.lru_cache(maxsize=None)
def _make_gather(n_flat: int, vocab: int, d: int):
    info = plsc.get_sparse_core_info()
    nw = info.num_cores * info.num_subcores
    assert n_flat % (nw * _CHUNK * _NBUF) == 0 and _CHUNK % 8 == 0
    b_per_w = n_flat // nw
    n_chunks = b_per_w // _CHUNK
    n_steps = n_chunks // _NBUF
    mesh = plsc.VectorSubcoreMesh(core_axis_name="c", subcore_axis_name="s")

    @functools.partial(
        pl.kernel,
        mesh=mesh,
        out_type=jax.ShapeDtypeStruct((n_flat // 8, 8, 2 * d), jnp.float32),
        scratch_types=[
            pltpu.VMEM((_NBUF, _CHUNK), jnp.int32),
            pltpu.VMEM((_NBUF, _CHUNK, d), jnp.float32),
            pltpu.SemaphoreType.DMA((_NBUF,)),
            pltpu.SemaphoreType.DMA((_NBUF,)),
        ],
        compiler_params=pltpu.CompilerParams(use_tc_tiling_on_sc=False),
    )
    def gather_kernel(idx_hbm, table_hbm, out_hbm, idx_v, rows_v, gsem, ssem):
        wid = lax.axis_index("s") * info.num_cores + lax.axis_index("c")
        base = wid * b_per_w

        def start_gather(ci, b):
            row0 = base + ci * _CHUNK
            pltpu.sync_copy(idx_hbm.at[pl.ds(row0, _CHUNK)], idx_v.at[b])
            pltpu.async_copy(table_hbm.at[idx_v.at[b]], rows_v.at[b],
                             gsem.at[b])

        def start_scatter(ci, b):
            g0 = (base + ci * _CHUNK) // 8
            for r in range(_CHUNK // 8):
                pltpu.async_copy(
                    rows_v.at[b, pl.ds(r * 8, 8)],
                    out_hbm.at[g0 + r, slice(None), pl.ds(0, d)],
                    ssem.at[b],
                )

        def wait_gather(ci, b):
            pltpu.make_async_copy(
                table_hbm.at[idx_v.at[b]], rows_v.at[b], gsem.at[b]
            ).wait()

        def wait_scatter(ci, b):
            g0 = (base + ci * _CHUNK) // 8
            for r in range(_CHUNK // 8):
                pltpu.make_async_copy(
                    rows_v.at[b, pl.ds(r * 8, 8)],
                    out_hbm.at[g0 + r, slice(None), pl.ds(0, d)],
                    ssem.at[b],
                ).wait()

        for b in range(_NBUF):
            start_gather(b, b)

        def body(g, carry):
            for b in range(_NBUF):
                ci = g * _NBUF + b
                wait_gather(ci, b)
                start_scatter(ci, b)
                wait_scatter(ci, b)
                start_gather(ci + _NBUF, b)
            return carry

        lax.fori_loop(0, n_steps - 1, body, 0)

        for b in range(_NBUF):
            ci = (n_steps - 1) * _NBUF + b
            wait_gather(ci, b)
            start_scatter(ci, b)
        for b in range(_NBUF):
            ci = (n_steps - 1) * _NBUF + b
            wait_scatter(ci, b)

    return gather_kernel


def kernel(input, table):
    b, l = input.shape
    vocab, d = table.shape
    n = b * l
    flat_idx = input.T.reshape(n)
    packed = _make_gather(n, vocab, d)(flat_idx, table)
    emb = packed.reshape(n, 2 * d)[:, :d].reshape(l, b, d)
    return emb.transpose(1, 0, 2)
